# baseline (speedup 1.0000x reference)
import functools

import jax
import jax.numpy as jnp
from jax import lax
from jax.experimental import pallas as pl
from jax.experimental.pallas import tpu as pltpu
from jax.experimental.pallas import tpu_sc as plsc

BLK = 512
LATS = 32
KSEL = 8
THRESH = 0.1
EPS = 1e-08
ONE_EPS = 1.0 + EPS
NC, NS, LANES = 2, 16, 16
NW = NC * NS


def _tc_encode(x_ref, w_ref, b_ref, d_ref, v_ref, i_ref, dc_ref):
    blk = x_ref.shape[0]
    zt = lax.dot_general(
        w_ref[...], x_ref[...], (((1,), (1,)), ((), ())),
        preferred_element_type=jnp.float32)
    zt = jax.nn.sigmoid(zt + b_ref[...])
    iota = lax.broadcasted_iota(jnp.int32, (LATS, blk), 0)
    zw = zt
    for t in range(KSEL):
        mx = jnp.max(zw, axis=0, keepdims=True)
        am = jnp.min(jnp.where(zw == mx, iota, LATS), axis=0, keepdims=True)
        zw = jnp.where(iota == am, -1.0, zw)
        v_ref[t:t + 1, :] = jnp.where(mx > THRESH, mx, 0.0)
        i_ref[t:t + 1, :] = am
    dc_ref[...] = jnp.clip(d_ref[...], 0.0, 1.0)


def _sc_decode_body(rpt, din, v_hbm, i_hbm, dc_hbm, o_hbm, v_v, i_v, d_v, o_v):
    wid = lax.axis_index("s") * NC + lax.axis_index("c")
    base = wid * rpt
    pltpu.sync_copy(dc_hbm, d_v)
    pltpu.sync_copy(v_hbm.at[:, pl.ds(base, rpt)], v_v)
    pltpu.sync_copy(i_hbm.at[:, pl.ds(base, rpt)], i_v)
    iota = lax.iota(jnp.int32, LANES)
    nchunk = din // LANES

    init = jnp.full((LANES,), ONE_EPS ** (LATS - KSEL), jnp.float32)

    @plsc.parallel_loop(0, rpt, 1, unroll=2)
    def row(r):
        rsp = jnp.zeros((LANES,), jnp.int32) + r
        vjs, tjs = [], []
        for j in range(KSEL):
            jv = jnp.full((LANES,), j, jnp.int32)
            vjs.append(plsc.load_gather(v_v, [jv, rsp]))
            ij = plsc.load_gather(i_v, [jv, rsp])
            tjs.append(ij * din + iota)
        for c in range(nchunk):
            acc = init
            for j in range(KSEL):
                dvec = plsc.load_gather(d_v, [tjs[j] + (c * LANES)])
                acc = acc * (ONE_EPS - vjs[j] * dvec)
            o_v[r, pl.ds(c * LANES, LANES)] = jnp.clip(
                1.0 - acc, 1e-07, 1.0 - 1e-07)
    pltpu.sync_copy(o_v, o_hbm.at[pl.ds(base, rpt), :])


@jax.jit
def kernel(x, W, b, D):
    batch, din = x.shape
    vt, it, dc = pl.pallas_call(
        _tc_encode,
        grid=(batch // BLK,),
        in_specs=[
            pl.BlockSpec((BLK, din), lambda i: (i, 0)),
            pl.BlockSpec((LATS, din), lambda i: (0, 0)),
            pl.BlockSpec((LATS, 1), lambda i: (0, 0)),
            pl.BlockSpec((LATS, din), lambda i: (0, 0)),
        ],
        out_specs=[
            pl.BlockSpec((KSEL, BLK), lambda i: (0, i)),
            pl.BlockSpec((KSEL, BLK), lambda i: (0, i)),
            pl.BlockSpec((LATS, din), lambda i: (0, 0)),
        ],
        out_shape=[
            jax.ShapeDtypeStruct((KSEL, batch), jnp.float32),
            jax.ShapeDtypeStruct((KSEL, batch), jnp.int32),
            jax.ShapeDtypeStruct((LATS, din), jnp.float32),
        ],
    )(x, W, b.reshape(LATS, 1), D)

    rpt = batch // NW
    mesh = plsc.VectorSubcoreMesh(
        core_axis_name="c", subcore_axis_name="s",
        num_cores=NC, num_subcores=NS)
    decode = functools.partial(
        pl.kernel,
        out_type=jax.ShapeDtypeStruct((batch, din), jnp.float32),
        mesh=mesh,
        compiler_params=pltpu.CompilerParams(needs_layout_passes=False),
        scratch_types=[
            pltpu.VMEM((KSEL, rpt), jnp.float32),
            pltpu.VMEM((KSEL, rpt), jnp.int32),
            pltpu.VMEM((LATS * din,), jnp.float32),
            pltpu.VMEM((rpt, din), jnp.float32),
        ],
    )(functools.partial(_sc_decode_body, rpt, din))
    return decode(vt, it, dc.reshape(-1))


# speedup vs baseline: 1.4340x; 1.4340x over previous
import functools

import jax
import jax.numpy as jnp
from jax import lax
from jax.experimental import pallas as pl
from jax.experimental.pallas import tpu as pltpu
from jax.experimental.pallas import tpu_sc as plsc

BLK = 512
LATS = 32
KSEL = 8
THRESH = 0.1
EPS = 1e-08
ONE_EPS = 1.0 + EPS
NC, NS, LANES = 2, 16, 16
NW = NC * NS


def _tc_encode(x_ref, w_ref, b_ref, d_ref, v_ref, i_ref, dc_ref):
    blk = x_ref.shape[0]
    zt = lax.dot_general(
        w_ref[...], x_ref[...], (((1,), (1,)), ((), ())),
        preferred_element_type=jnp.float32)
    zt = jax.nn.sigmoid(zt + b_ref[...])
    iota = lax.broadcasted_iota(jnp.int32, (LATS, blk), 0)
    zw = zt
    for t in range(KSEL):
        mx = jnp.max(zw, axis=0, keepdims=True)
        am = jnp.min(jnp.where(zw == mx, iota, LATS), axis=0, keepdims=True)
        zw = jnp.where(iota == am, -1.0, zw)
        v_ref[t:t + 1, :] = jnp.where(mx > THRESH, mx, 0.0)
        i_ref[t:t + 1, :] = am
    dc_ref[...] = jnp.clip(d_ref[...], 0.0, 1.0)


def _sc_decode_body(rpt, din, v_hbm, i_hbm, dc_hbm, o_hbm, v_v, i_v, d_v, o_v):
    wid = lax.axis_index("s") * NC + lax.axis_index("c")
    base = wid * rpt
    pltpu.sync_copy(dc_hbm, d_v)
    pltpu.sync_copy(v_hbm.at[:, pl.ds(base, rpt)], v_v)
    pltpu.sync_copy(i_hbm.at[:, pl.ds(base, rpt)], i_v)
    iota = lax.iota(jnp.int32, LANES)
    nchunk = din // LANES

    init = jnp.full((LANES,), ONE_EPS ** (LATS - KSEL), jnp.float32)

    @plsc.parallel_loop(0, rpt, 1, unroll=2)
    def row(r):
        rsp = jnp.zeros((LANES,), jnp.int32) + r
        vjs, ijs = [], []
        for j in range(KSEL):
            jv = jnp.full((LANES,), j, jnp.int32)
            vjs.append(plsc.load_gather(v_v, [jv, rsp]))
            ijs.append(plsc.load_gather(i_v, [jv, rsp]))
        tjs = [ij * din + iota for ij in ijs]
        accs = [init] * nchunk
        for j in range(KSEL):
            for c in range(nchunk):
                dvec = plsc.load_gather(
                    d_v.at[pl.ds(c * LANES, LATS * din - c * LANES)], [tjs[j]])
                accs[c] = accs[c] * (ONE_EPS - vjs[j] * dvec)
        for c in range(nchunk):
            o_v[r, pl.ds(c * LANES, LANES)] = jnp.clip(
                1.0 - accs[c], 1e-07, 1.0 - 1e-07)
    pltpu.sync_copy(o_v, o_hbm.at[pl.ds(base, rpt), :])


@jax.jit
def kernel(x, W, b, D):
    batch, din = x.shape
    vt, it, dc = pl.pallas_call(
        _tc_encode,
        grid=(batch // BLK,),
        in_specs=[
            pl.BlockSpec((BLK, din), lambda i: (i, 0)),
            pl.BlockSpec((LATS, din), lambda i: (0, 0)),
            pl.BlockSpec((LATS, 1), lambda i: (0, 0)),
            pl.BlockSpec((LATS, din), lambda i: (0, 0)),
        ],
        out_specs=[
            pl.BlockSpec((KSEL, BLK), lambda i: (0, i)),
            pl.BlockSpec((KSEL, BLK), lambda i: (0, i)),
            pl.BlockSpec((LATS, din), lambda i: (0, 0)),
        ],
        out_shape=[
            jax.ShapeDtypeStruct((KSEL, batch), jnp.float32),
            jax.ShapeDtypeStruct((KSEL, batch), jnp.int32),
            jax.ShapeDtypeStruct((LATS, din), jnp.float32),
        ],
    )(x, W, b.reshape(LATS, 1), D)

    rpt = batch // NW
    mesh = plsc.VectorSubcoreMesh(
        core_axis_name="c", subcore_axis_name="s",
        num_cores=NC, num_subcores=NS)
    decode = functools.partial(
        pl.kernel,
        out_type=jax.ShapeDtypeStruct((batch, din), jnp.float32),
        mesh=mesh,
        compiler_params=pltpu.CompilerParams(needs_layout_passes=False),
        scratch_types=[
            pltpu.VMEM((KSEL, rpt), jnp.float32),
            pltpu.VMEM((KSEL, rpt), jnp.int32),
            pltpu.VMEM((LATS * din,), jnp.float32),
            pltpu.VMEM((rpt, din), jnp.float32),
        ],
    )(functools.partial(_sc_decode_body, rpt, din))
    return decode(vt, it, dc.reshape(-1))
